# baseline (speedup 1.0000x reference)
import jax
import jax.numpy as jnp
from jax import lax
from jax.experimental import pallas as pl
from jax.experimental.pallas import tpu as pltpu
from jax.experimental.pallas import tpu_sc as plsc

_MAGIC = 12582912.0

N_ROWS = 1048576
NW = 32
ROWS_PER_W = N_ROWS // NW
CHUNK = 4096
NCHUNK = ROWS_PER_W // CHUNK
GROUPS = CHUNK // 16
CW = CHUNK * 8


def _rne(x):
    return (x + _MAGIC) - _MAGIC


def _decode8(zs):
    fs, ds, absds = [], [], []
    sum_f = sum_d2 = m = None
    for z in zs:
        f = _rne(z)
        d = z - f
        a = jnp.abs(d)
        d2 = d * d
        sum_f = f if sum_f is None else sum_f + f
        sum_d2 = d2 if sum_d2 is None else sum_d2 + d2
        m = a if m is None else jnp.maximum(m, a)
        fs.append(f)
        ds.append(d)
        absds.append(a)
    h = sum_f * 0.5
    odd = _rne(h) != h
    dist = sum_d2 + jnp.where(odd, 1.0 - (m + m), 0.0)
    modd = jnp.where(odd, m, -1.0)
    gs = []
    for f, d, a in zip(fs, ds, absds):
        stp = jnp.where(d >= 0, 1.0, -1.0)
        gs.append(f + jnp.where(a == modd, stp, 0.0))
    return gs, dist


def _sc_kernel(x_hbm, o_hbm, x_v, o_v):
    cid = lax.axis_index("c")
    sid = lax.axis_index("s")
    wid = sid * 2 + cid
    iota16 = lax.iota(jnp.int32, 16)
    base8 = [iota16 * 8 + j for j in range(8)]

    for c in range(NCHUNK):
        base = wid * (ROWS_PER_W * 8) + c * CW
        pltpu.sync_copy(x_hbm.at[pl.ds(base, CW)], x_v)

        def body(g, carry):
            g128 = g * 128
            idx = [g128 + base8[j] for j in range(8)]
            xs = [plsc.load_gather(x_v, [idx[j]]) for j in range(8)]
            g0s, d0 = _decode8(xs)
            g1s, d1 = _decode8([x - 0.5 for x in xs])
            ch = d0 <= d1
            for j in range(8):
                y = jnp.where(ch, g0s[j], g1s[j] + 0.5)
                plsc.store_scatter(o_v, [idx[j]], y)
            return carry

        lax.fori_loop(0, GROUPS, body, 0)
        pltpu.sync_copy(o_v, o_hbm.at[pl.ds(base, CW)])


@jax.jit
def _e8_sc(x):
    xf = jnp.reshape(x, (N_ROWS * 8,))
    mesh = plsc.VectorSubcoreMesh(core_axis_name="c", subcore_axis_name="s")
    f = pl.kernel(
        _sc_kernel,
        mesh=mesh,
        out_type=jax.ShapeDtypeStruct((N_ROWS * 8,), jnp.float32),
        scratch_types=[
            pltpu.VMEM((CW,), jnp.float32),
            pltpu.VMEM((CW,), jnp.float32),
        ],
        compiler_params=pltpu.CompilerParams(needs_layout_passes=False),
    )
    return jnp.reshape(f(xf), (N_ROWS, 8))


def kernel(x):
    return _e8_sc(x)
